# baseline (speedup 1.0000x reference)
import numpy as np

import jax
import jax.numpy as jnp
from jax.experimental import pallas as pl
from jax.experimental.pallas import tpu as pltpu

_H, _W = 128, 256
_L, _C = 5, 64
_K = (_H * _W) // 2


def _gauss_weights(k_size=5, sigma=1.0):
    center = k_size // 2
    gx, gy = np.mgrid[0 - center:k_size - center, 0 - center:k_size - center]
    g = 1.0 / (2.0 * np.pi * sigma) * np.exp(-(np.square(gx) + np.square(gy)) / (2.0 * np.square(sigma)))
    return g.astype(np.float32)


_GW = _gauss_weights()
_GWB = _GW.astype(jnp.bfloat16).astype(np.float32)


def _band_matrices():
    B = np.zeros((5, _W, _W), np.float32)
    for dy in range(5):
        for d in range(-2, 3):
            for x in range(_W):
                k = x + d
                if 0 <= k < _W:
                    B[dy, k, x] = _GWB[dy, d + 2]
    return B


_BNP = _band_matrices()


def _select_map(conf_ref, b_ref, m):
    H, W, K = _H, _W, _K
    s = jnp.maximum(jax.nn.sigmoid(conf_ref[m, 0]), jax.nn.sigmoid(conf_ref[m, 1]))
    sb = s.astype(jnp.bfloat16).astype(jnp.float32)
    zrow = jnp.zeros((2, W), jnp.float32)
    sp = jnp.concatenate([zrow, sb, zrow], axis=0)
    acc = jnp.zeros((H, W), jnp.float32)
    for dy in range(5):
        sv = jax.lax.slice(sp, (dy, 0), (dy + H, W)).astype(jnp.bfloat16)
        acc = acc + jax.lax.dot_general(
            sv, b_ref[dy], (((1,), (0,)), ((), ())),
            preferred_element_type=jnp.float32)
    keys = jax.lax.bitcast_convert_type(acc, jnp.int32)
    prefix = jnp.zeros((1, 1), jnp.int32)
    for hi in range(29, -1, -2):
        lo = hi - 1
        c10 = prefix | (1 << hi)
        c01 = prefix | (1 << lo)
        c11 = c10 | (1 << lo)
        n10 = jnp.sum((keys >= c10).astype(jnp.int32), keepdims=True)
        n01 = jnp.sum((keys >= c01).astype(jnp.int32), keepdims=True)
        n11 = jnp.sum((keys >= c11).astype(jnp.int32), keepdims=True)
        prefix = jnp.where(n10 >= K,
                           jnp.where(n11 >= K, c11, c10),
                           jnp.where(n01 >= K, c01, prefix))
    thresh = prefix
    gcnt = jnp.sum((keys > thresh).astype(jnp.int32), keepdims=True)
    need = K - gcnt
    tie = keys == thresh
    fidx = (jax.lax.broadcasted_iota(jnp.int32, (H, W), 0) * W
            + jax.lax.broadcasted_iota(jnp.int32, (H, W), 1))
    P = jnp.zeros((1, 1), jnp.int32)
    for bit in range(14, -1, -1):
        mid = P | (1 << bit)
        cnt = jnp.sum((tie & (fidx < mid)).astype(jnp.int32), keepdims=True)
        P = jnp.where(cnt >= need, P, mid)
    return ((keys > thresh) | (tie & (fidx <= P))).astype(jnp.float32)


def _gt_and_rate(tgt_ref, rate_ref, mgs):
    H, W = _H, _W
    ys = jax.lax.broadcasted_iota(jnp.int32, (H, W), 0)
    xs = jax.lax.broadcasted_iota(jnp.int32, (H, W), 1)
    gt = jnp.zeros((H, W), jnp.bool_)
    for i in range(10):
        x1 = jnp.maximum(tgt_ref[i, 0], 0)
        y1 = jnp.maximum(tgt_ref[i, 1], 0)
        x2 = jnp.minimum(tgt_ref[i, 2], W)
        y2 = jnp.minimum(tgt_ref[i, 3], H)
        gt = gt | ((ys >= y1) & (ys < y2) & (xs >= x1) & (xs < x2))
    gtf = gt.astype(jnp.float32)
    mgs[...] = gtf
    rate_ref[0, 0] = 0.5 + jnp.sum(gtf) / float(H * W)


_CB = 32


def _fused(conf_ref, tgt_ref, b_ref, x_ref, oc_ref, og_ref, rate_ref, mcs, mgs):
    l = pl.program_id(0)
    c = pl.program_id(1)
    xv = x_ref[...]

    @pl.when(l == 0)
    def _():
        oc_ref[...] = xv
        og_ref[...] = xv

    @pl.when(l > 0)
    def _():
        m = mcs[pl.ds(l - 1, 1), :, :]
        oc_ref[...] = xv * m[None]
        og_ref[...] = xv * mgs[...][None, None]

    for m in range(1, 5):
        @pl.when((l == m - 1) & (c == 0))
        def _(m=m):
            mcs[m - 1] = _select_map(conf_ref, b_ref, m)

    @pl.when((l == 0) & (c == 1))
    def _():
        _gt_and_rate(tgt_ref, rate_ref, mgs)


def kernel(x, confidence_maps, targets_label, B):
    H, W, L, C, Cb = _H, _W, _L, _C, _CB
    xc, xg, rate = pl.pallas_call(
        _fused,
        grid=(L, C // Cb),
        in_specs=[
            pl.BlockSpec(memory_space=pltpu.VMEM),
            pl.BlockSpec(memory_space=pltpu.SMEM),
            pl.BlockSpec(memory_space=pltpu.VMEM),
            pl.BlockSpec((1, Cb, H, W), lambda l, c: (l, c, 0, 0)),
        ],
        out_specs=(
            pl.BlockSpec((1, Cb, H, W), lambda l, c: (l, c, 0, 0)),
            pl.BlockSpec((1, Cb, H, W), lambda l, c: (l, c, 0, 0)),
            pl.BlockSpec(memory_space=pltpu.SMEM),
        ),
        out_shape=(
            jax.ShapeDtypeStruct((L, C, H, W), jnp.float32),
            jax.ShapeDtypeStruct((L, C, H, W), jnp.float32),
            jax.ShapeDtypeStruct((1, 1), jnp.float32),
        ),
        scratch_shapes=[
            pltpu.VMEM((4, H, W), jnp.float32),
            pltpu.VMEM((H, W), jnp.float32),
        ],
    )(confidence_maps, targets_label, jnp.asarray(_BNP, jnp.bfloat16), x)
    return xc, xg, rate[0, 0]


# speedup vs baseline: 1.1144x; 1.1144x over previous
import numpy as np

import jax
import jax.numpy as jnp
from jax.experimental import pallas as pl
from jax.experimental.pallas import tpu as pltpu

_H, _W = 128, 256
_L, _C = 5, 64
_K = (_H * _W) // 2


def _gauss_weights(k_size=5, sigma=1.0):
    center = k_size // 2
    gx, gy = np.mgrid[0 - center:k_size - center, 0 - center:k_size - center]
    g = 1.0 / (2.0 * np.pi * sigma) * np.exp(-(np.square(gx) + np.square(gy)) / (2.0 * np.square(sigma)))
    return g.astype(np.float32)


_GW = _gauss_weights()
_GWB = _GW.astype(jnp.bfloat16).astype(np.float32)


def _band_matrices():
    B = np.zeros((5, _W, _W), np.float32)
    for dy in range(5):
        for d in range(-2, 3):
            for x in range(_W):
                k = x + d
                if 0 <= k < _W:
                    B[dy, k, x] = _GWB[dy, d + 2]
    return B


_BNP = _band_matrices()


def _compute_masks(conf_ref, tgt_ref, b_ref, rate_ref, mcs, mgs):
    H, W, K = _H, _W, _K
    c = conf_ref[...]
    s = jnp.maximum(jax.nn.sigmoid(c[:, 0]), jax.nn.sigmoid(c[:, 1]))
    sb = s[1:5].astype(jnp.bfloat16).astype(jnp.float32)
    zrow = jnp.zeros((4, 2, W), jnp.float32)
    sp = jnp.concatenate([zrow, sb, zrow], axis=1)
    acc = jnp.zeros((4, H, W), jnp.float32)
    for dy in range(5):
        sv = jax.lax.slice(sp, (0, dy, 0), (4, dy + H, W)).astype(jnp.bfloat16)
        acc = acc + jax.lax.dot_general(
            sv, b_ref[dy], (((2,), (0,)), ((), ())),
            preferred_element_type=jnp.float32)
    keys = jax.lax.bitcast_convert_type(acc, jnp.int32)
    prefix = jnp.zeros((4, 1, 1), jnp.int32)
    for hi in range(29, -1, -3):
        cands = [prefix | (j << (hi - 2)) for j in range(1, 8)]
        cnts = [jnp.sum((keys >= cd).astype(jnp.int32), axis=(1, 2), keepdims=True)
                for cd in cands]
        for cd, n in zip(cands, cnts):
            prefix = jnp.where(n >= K, cd, prefix)
    thresh = prefix
    gcnt = jnp.sum((keys > thresh).astype(jnp.int32), axis=(1, 2), keepdims=True)
    need = K - gcnt
    tie = keys == thresh
    fidx = (jax.lax.broadcasted_iota(jnp.int32, (H, W), 0) * W
            + jax.lax.broadcasted_iota(jnp.int32, (H, W), 1))[None]
    P = jnp.zeros((4, 1, 1), jnp.int32)
    for bit in range(14, -1, -1):
        mid = P | (1 << bit)
        cnt = jnp.sum((tie & (fidx < mid)).astype(jnp.int32), axis=(1, 2), keepdims=True)
        P = jnp.where(cnt >= need, P, mid)
    mcs[...] = ((keys > thresh) | (tie & (fidx <= P))).astype(jnp.float32)

    ys = jax.lax.broadcasted_iota(jnp.int32, (H, W), 0)
    xs = jax.lax.broadcasted_iota(jnp.int32, (H, W), 1)
    gt = jnp.zeros((H, W), jnp.bool_)
    for i in range(10):
        x1 = jnp.maximum(tgt_ref[i, 0], 0)
        y1 = jnp.maximum(tgt_ref[i, 1], 0)
        x2 = jnp.minimum(tgt_ref[i, 2], W)
        y2 = jnp.minimum(tgt_ref[i, 3], H)
        gt = gt | ((ys >= y1) & (ys < y2) & (xs >= x1) & (xs < x2))
    gtf = gt.astype(jnp.float32)
    mgs[...] = gtf
    rate_ref[0, 0] = 0.5 + jnp.sum(gtf) / float(H * W)


_CB = 32


def _fused(conf_ref, tgt_ref, b_ref, x_ref, oc_ref, og_ref, rate_ref, mcs, mgs):
    l = pl.program_id(0)
    c = pl.program_id(1)
    xv = x_ref[...]

    @pl.when(l == 0)
    def _():
        oc_ref[...] = xv
        og_ref[...] = xv

    @pl.when((l == 0) & (c == 0))
    def _():
        _compute_masks(conf_ref, tgt_ref, b_ref, rate_ref, mcs, mgs)

    @pl.when(l > 0)
    def _():
        m = mcs[pl.ds(l - 1, 1), :, :]
        oc_ref[...] = xv * m[None]
        og_ref[...] = xv * mgs[...][None, None]


def kernel(x, confidence_maps, targets_label, B):
    H, W, L, C, Cb = _H, _W, _L, _C, _CB
    xc, xg, rate = pl.pallas_call(
        _fused,
        grid=(L, C // Cb),
        in_specs=[
            pl.BlockSpec(memory_space=pltpu.VMEM),
            pl.BlockSpec(memory_space=pltpu.SMEM),
            pl.BlockSpec(memory_space=pltpu.VMEM),
            pl.BlockSpec((1, Cb, H, W), lambda l, c: (l, c, 0, 0)),
        ],
        out_specs=(
            pl.BlockSpec((1, Cb, H, W), lambda l, c: (l, c, 0, 0)),
            pl.BlockSpec((1, Cb, H, W), lambda l, c: (l, c, 0, 0)),
            pl.BlockSpec(memory_space=pltpu.SMEM),
        ),
        out_shape=(
            jax.ShapeDtypeStruct((L, C, H, W), jnp.float32),
            jax.ShapeDtypeStruct((L, C, H, W), jnp.float32),
            jax.ShapeDtypeStruct((1, 1), jnp.float32),
        ),
        scratch_shapes=[
            pltpu.VMEM((4, H, W), jnp.float32),
            pltpu.VMEM((H, W), jnp.float32),
        ],
    )(confidence_maps, targets_label, jnp.asarray(_BNP, jnp.bfloat16), x)
    return xc, xg, rate[0, 0]


# speedup vs baseline: 1.1479x; 1.0301x over previous
import numpy as np

import jax
import jax.numpy as jnp
from jax.experimental import pallas as pl
from jax.experimental.pallas import tpu as pltpu

_H, _W = 128, 256
_L, _C = 5, 64
_K = (_H * _W) // 2


def _gauss_weights(k_size=5, sigma=1.0):
    center = k_size // 2
    gx, gy = np.mgrid[0 - center:k_size - center, 0 - center:k_size - center]
    g = 1.0 / (2.0 * np.pi * sigma) * np.exp(-(np.square(gx) + np.square(gy)) / (2.0 * np.square(sigma)))
    return g.astype(np.float32)


_GW = _gauss_weights()
_GWB = _GW.astype(jnp.bfloat16).astype(np.float32)


def _band_matrices():
    B = np.zeros((5, _W, _W), np.float32)
    for dy in range(5):
        for d in range(-2, 3):
            for x in range(_W):
                k = x + d
                if 0 <= k < _W:
                    B[dy, k, x] = _GWB[dy, d + 2]
    return B


_BNP = _band_matrices()


def _compute_masks(conf_ref, tgt_ref, b_ref, rate_ref, mcs, mgs):
    H, W, K = _H, _W, _K
    c = conf_ref[...]
    s = jnp.maximum(jax.nn.sigmoid(c[:, 0]), jax.nn.sigmoid(c[:, 1]))
    sb = s[1:5].astype(jnp.bfloat16).astype(jnp.float32)
    zrow = jnp.zeros((4, 2, W), jnp.float32)
    sp = jnp.concatenate([zrow, sb, zrow], axis=1)
    acc = jnp.zeros((4, H, W), jnp.float32)
    for dy in range(5):
        sv = jax.lax.slice(sp, (0, dy, 0), (4, dy + H, W)).astype(jnp.bfloat16)
        acc = acc + jax.lax.dot_general(
            sv, b_ref[dy], (((2,), (0,)), ((), ())),
            preferred_element_type=jnp.float32)
    keys = jax.lax.bitcast_convert_type(acc, jnp.int32)
    prefix = jnp.zeros((4, 1, 1), jnp.int32)
    for hi in range(29, -1, -2):
        lo = hi - 1
        c10 = prefix | (1 << hi)
        c01 = prefix | (1 << lo)
        c11 = c10 | (1 << lo)
        n10 = jnp.sum((keys >= c10).astype(jnp.int32), axis=(1, 2), keepdims=True)
        n01 = jnp.sum((keys >= c01).astype(jnp.int32), axis=(1, 2), keepdims=True)
        n11 = jnp.sum((keys >= c11).astype(jnp.int32), axis=(1, 2), keepdims=True)
        prefix = jnp.where(n10 >= K,
                           jnp.where(n11 >= K, c11, c10),
                           jnp.where(n01 >= K, c01, prefix))
    thresh = prefix
    gcnt = jnp.sum((keys > thresh).astype(jnp.int32), axis=(1, 2), keepdims=True)
    need = K - gcnt
    tie = keys == thresh
    fidx = (jax.lax.broadcasted_iota(jnp.int32, (H, W), 0) * W
            + jax.lax.broadcasted_iota(jnp.int32, (H, W), 1))[None]
    P = jnp.zeros((4, 1, 1), jnp.int32)
    for bit in range(14, -1, -1):
        mid = P | (1 << bit)
        cnt = jnp.sum((tie & (fidx < mid)).astype(jnp.int32), axis=(1, 2), keepdims=True)
        P = jnp.where(cnt >= need, P, mid)
    mcs[...] = ((keys > thresh) | (tie & (fidx <= P))).astype(jnp.float32)

    ys = jax.lax.broadcasted_iota(jnp.int32, (H, W), 0)
    xs = jax.lax.broadcasted_iota(jnp.int32, (H, W), 1)
    gt = jnp.zeros((H, W), jnp.bool_)
    for i in range(10):
        x1 = jnp.maximum(tgt_ref[i, 0], 0)
        y1 = jnp.maximum(tgt_ref[i, 1], 0)
        x2 = jnp.minimum(tgt_ref[i, 2], W)
        y2 = jnp.minimum(tgt_ref[i, 3], H)
        gt = gt | ((ys >= y1) & (ys < y2) & (xs >= x1) & (xs < x2))
    gtf = gt.astype(jnp.float32)
    mgs[...] = gtf
    rate_ref[0, 0] = 0.5 + jnp.sum(gtf) / float(H * W)


_CB = 32


def _fused(conf_ref, tgt_ref, b_ref, x_ref, oc_ref, og_ref, rate_ref, mcs, mgs):
    l = pl.program_id(0)
    c = pl.program_id(1)
    xv = x_ref[...]

    @pl.when(l == 0)
    def _():
        oc_ref[...] = xv
        og_ref[...] = xv

    @pl.when((l == 0) & (c == 0))
    def _():
        _compute_masks(conf_ref, tgt_ref, b_ref, rate_ref, mcs, mgs)

    @pl.when(l > 0)
    def _():
        m = mcs[pl.ds(l - 1, 1), :, :]
        oc_ref[...] = xv * m[None]
        og_ref[...] = xv * mgs[...][None, None]


def kernel(x, confidence_maps, targets_label, B):
    H, W, L, C, Cb = _H, _W, _L, _C, _CB
    xc, xg, rate = pl.pallas_call(
        _fused,
        grid=(L, C // Cb),
        in_specs=[
            pl.BlockSpec(memory_space=pltpu.VMEM),
            pl.BlockSpec(memory_space=pltpu.SMEM),
            pl.BlockSpec(memory_space=pltpu.VMEM),
            pl.BlockSpec((1, Cb, H, W), lambda l, c: (l, c, 0, 0)),
        ],
        out_specs=(
            pl.BlockSpec((1, Cb, H, W), lambda l, c: (l, c, 0, 0)),
            pl.BlockSpec((1, Cb, H, W), lambda l, c: (l, c, 0, 0)),
            pl.BlockSpec(memory_space=pltpu.SMEM),
        ),
        out_shape=(
            jax.ShapeDtypeStruct((L, C, H, W), jnp.float32),
            jax.ShapeDtypeStruct((L, C, H, W), jnp.float32),
            jax.ShapeDtypeStruct((1, 1), jnp.float32),
        ),
        scratch_shapes=[
            pltpu.VMEM((4, H, W), jnp.float32),
            pltpu.VMEM((H, W), jnp.float32),
        ],
    )(confidence_maps, targets_label, jnp.asarray(_BNP, jnp.bfloat16), x)
    return xc, xg, rate[0, 0]
